# baseline (speedup 1.0000x reference)
import functools

import jax
import jax.numpy as jnp
from jax import lax
from jax.experimental import pallas as pl
from jax.experimental.pallas import tpu as pltpu
from jax.experimental.pallas import tpu_sc as plsc

_OUT_H = 7
_OUT_W = 7
_SCALE = 0.25
_SN = 2
_E = _SN * _SN * 4
_PB = _OUT_H * _OUT_W

_NW = 32
_CH_BINS = 16
_ROWS = _CH_BINS * _E


def _prep_math(r, H, W):
    n = r.shape[0]
    shp = (n, _PB * _E)
    e = lax.broadcasted_iota(jnp.int32, shp, 1)
    corner = e % 4
    s = (e // 4) % (_SN * _SN)
    sx = (s % _SN).astype(jnp.float32)
    sy = (s // _SN).astype(jnp.float32)
    b = e // _E
    pw = (b % _OUT_W).astype(jnp.float32)
    ph = (b // _OUT_W).astype(jnp.float32)

    bidx = r[:, 0:1].astype(jnp.int32)
    cx = r[:, 1:2] * _SCALE
    cy = r[:, 2:3] * _SCALE
    rw = jnp.maximum(r[:, 3:4] * _SCALE, 1.0)
    rh = jnp.maximum(r[:, 4:5] * _SCALE, 1.0)
    th = r[:, 5:6]

    bin_w = rw / _OUT_W
    bin_h = rh / _OUT_H
    xl = -rw * 0.5 + pw * bin_w + (sx + 0.5) * bin_w / _SN
    yl = -rh * 0.5 + ph * bin_h + (sy + 0.5) * bin_h / _SN
    ct = jnp.cos(th)
    st = jnp.sin(th)
    x = xl * ct - yl * st + cx
    y = xl * st + yl * ct + cy

    valid = (y > -1.0) & (y < H) & (x > -1.0) & (x < W)
    y = jnp.maximum(y, 0.0)
    x = jnp.maximum(x, 0.0)
    y_low = jnp.floor(y).astype(jnp.int32)
    x_low = jnp.floor(x).astype(jnp.int32)
    y_hi = y_low >= H - 1
    x_hi = x_low >= W - 1
    y_low = jnp.where(y_hi, H - 1, y_low)
    x_low = jnp.where(x_hi, W - 1, x_low)
    y_high = jnp.where(y_hi, H - 1, y_low + 1)
    x_high = jnp.where(x_hi, W - 1, x_low + 1)
    y = jnp.where(y_hi, y_low.astype(jnp.float32), y)
    x = jnp.where(x_hi, x_low.astype(jnp.float32), x)
    ly = y - y_low.astype(jnp.float32)
    lx = x - x_low.astype(jnp.float32)
    hy = 1.0 - ly
    hx = 1.0 - lx

    wy = jnp.where(corner < 2, hy, ly)
    wx = jnp.where(corner % 2 == 0, hx, lx)
    ysel = jnp.where(corner < 2, y_low, y_high)
    xsel = jnp.where(corner % 2 == 0, x_low, x_high)

    idx = bidx * (H * W) + ysel * W + xsel
    wgt = jnp.where(valid, wy * wx * (1.0 / (_SN * _SN)), 0.0)
    return idx, wgt


def _prep(rois, H, W):
    N = rois.shape[0]
    nblk = 64

    def body(rois_ref, idx_ref, wgt_ref):
        idx, wgt = _prep_math(rois_ref[...], H, W)
        idx_ref[...] = idx
        wgt_ref[...] = wgt

    return pl.pallas_call(
        body,
        grid=(N // nblk,),
        in_specs=[pl.BlockSpec((nblk, 6), lambda i: (i, 0))],
        out_specs=[
            pl.BlockSpec((nblk, _PB * _E), lambda i: (i, 0)),
            pl.BlockSpec((nblk, _PB * _E), lambda i: (i, 0)),
        ],
        out_shape=[
            jax.ShapeDtypeStruct((N, _PB * _E), jnp.int32),
            jax.ShapeDtypeStruct((N, _PB * _E), jnp.float32),
        ],
    )(rois)


def _sc_pool(table, idx_flat, wgt_flat, tot_bins):
    C = table.shape[1] * 2
    bins_w = tot_bins // _NW
    n_chunks = bins_w // _CH_BINS
    nvec = C // 16

    ew = bins_w * _E

    mesh = plsc.VectorSubcoreMesh(core_axis_name="c", subcore_axis_name="s")

    @functools.partial(
        pl.kernel,
        mesh=mesh,
        out_type=jax.ShapeDtypeStruct((tot_bins, C), jnp.float32),
        scratch_types=[
            pltpu.VMEM((ew,), jnp.int32),
            pltpu.VMEM((ew,), jnp.float32),
            pltpu.VMEM((_ROWS, C // 2), jnp.int32),
            pltpu.VMEM((_ROWS, C // 2), jnp.int32),
            pltpu.VMEM((_CH_BINS, C), jnp.float32),
            pltpu.SemaphoreType.DMA,
            pltpu.SemaphoreType.DMA,
        ],
        compiler_params=pltpu.CompilerParams(needs_layout_passes=False),
    )
    def body(table_hbm, idx_hbm, wgt_hbm, out_hbm,
             idx_all, wgt_all, rows0, rows1, out_v, sem0, sem1):
        wid = lax.axis_index("s") * 2 + lax.axis_index("c")
        rows = (rows0, rows1)
        sems = (sem0, sem1)

        pltpu.sync_copy(idx_hbm.at[pl.ds(wid * ew, ew)], idx_all)
        pltpu.sync_copy(wgt_hbm.at[pl.ds(wid * ew, ew)], wgt_all)

        def start_gather(ci, b):
            pltpu.async_copy(
                table_hbm.at[idx_all.at[pl.ds(ci * _ROWS, _ROWS)]],
                rows[b], sems[b])

        def wait_gather(ci, b):
            pltpu.make_async_copy(
                table_hbm.at[idx_all.at[pl.ds(ci * _ROWS, _ROWS)]],
                rows[b], sems[b]).wait()

        def compute(ci, b):
            rows_v = rows[b]
            gbin = wid * bins_w + ci * _CH_BINS

            @plsc.parallel_loop(0, _CH_BINS, unroll=2)
            def bin_body(bb):
                rbase = bb * _E
                wv = wgt_all[pl.ds(ci * _ROWS + rbase, _E)]
                accs = [jnp.zeros((32,), jnp.bfloat16) for _ in range(nvec // 2)]
                for j in range(_E):
                    wf = jnp.broadcast_to(wv[j], (16,))
                    wb = plsc.pack(wf, wf, format=plsc.PackFormat.INTERLEAVED)
                    rr = rbase + j
                    for t in range(nvec // 2):
                        vb = plsc.bitcast(
                            rows_v[rr, pl.ds(t * 16, 16)], jnp.bfloat16)
                        accs[t] = accs[t] + wb * vb
                for t in range(nvec // 2):
                    ai = plsc.bitcast(accs[t], jnp.int32)
                    lo = plsc.bitcast(ai << 16, jnp.float32)
                    hi = plsc.bitcast(ai & jnp.int32(-65536), jnp.float32)
                    out_v[bb, pl.ds((2 * t) * 16, 16)] = lo
                    out_v[bb, pl.ds((2 * t + 1) * 16, 16)] = hi

            pltpu.sync_copy(out_v, out_hbm.at[pl.ds(gbin, _CH_BINS)])

        start_gather(0, 0)
        start_gather(1, 1)

        def outer(io, carry):
            for b in range(2):
                ci = io * 2 + b
                wait_gather(ci, b)
                compute(ci, b)
                start_gather(ci + 2, b)
            return carry

        epi = 3 if n_chunks % 2 else 2
        lax.fori_loop(0, (n_chunks - epi) // 2, outer, 0)
        for ci_ in range(n_chunks - epi, n_chunks):
            b = ci_ % 2
            wait_gather(ci_, b)
            compute(ci_, b)
            if ci_ + 2 < n_chunks:
                start_gather(ci_ + 2, b)

    return body(table, idx_flat, wgt_flat)


def kernel(features, rois):
    B, C, H, W = features.shape
    N = rois.shape[0]
    tb = jnp.transpose(features, (0, 2, 3, 1)).reshape(
        B * H * W, C).astype(jnp.bfloat16)
    u = jax.lax.bitcast_convert_type(tb, jnp.uint16).reshape(-1, C // 32, 2, 16)
    ti = (u[:, :, 0, :].astype(jnp.int32)
          | (u[:, :, 1, :].astype(jnp.int32) << 16)).reshape(-1, C // 2)
    idx, wgt = _prep(rois, H, W)
    out = _sc_pool(ti, idx.reshape(-1), wgt.reshape(-1), N * _PB)
    return out.reshape(N, _PB, C).transpose(0, 2, 1).reshape(N, C, _OUT_H, _OUT_W)


# speedup vs baseline: 1.0474x; 1.0474x over previous
import functools

import jax
import jax.numpy as jnp
from jax import lax
from jax.experimental import pallas as pl
from jax.experimental.pallas import tpu as pltpu
from jax.experimental.pallas import tpu_sc as plsc

_OUT_H = 7
_OUT_W = 7
_SCALE = 0.25
_SN = 2
_E = _SN * _SN * 4
_PB = _OUT_H * _OUT_W

_NW = 32
_CH_BINS = 16
_ROWS = _CH_BINS * _E


def _prep_math(r, H, W):
    n = r.shape[0]
    shp = (n, _PB * _E)
    e = lax.broadcasted_iota(jnp.int32, shp, 1)
    corner = e % 4
    s = (e // 4) % (_SN * _SN)
    sx = (s % _SN).astype(jnp.float32)
    sy = (s // _SN).astype(jnp.float32)
    b = e // _E
    pw = (b % _OUT_W).astype(jnp.float32)
    ph = (b // _OUT_W).astype(jnp.float32)

    bidx = r[:, 0:1].astype(jnp.int32)
    cx = r[:, 1:2] * _SCALE
    cy = r[:, 2:3] * _SCALE
    rw = jnp.maximum(r[:, 3:4] * _SCALE, 1.0)
    rh = jnp.maximum(r[:, 4:5] * _SCALE, 1.0)
    th = r[:, 5:6]

    bin_w = rw / _OUT_W
    bin_h = rh / _OUT_H
    xl = -rw * 0.5 + pw * bin_w + (sx + 0.5) * bin_w / _SN
    yl = -rh * 0.5 + ph * bin_h + (sy + 0.5) * bin_h / _SN
    ct = jnp.cos(th)
    st = jnp.sin(th)
    x = xl * ct - yl * st + cx
    y = xl * st + yl * ct + cy

    valid = (y > -1.0) & (y < H) & (x > -1.0) & (x < W)
    y = jnp.maximum(y, 0.0)
    x = jnp.maximum(x, 0.0)
    y_low = jnp.floor(y).astype(jnp.int32)
    x_low = jnp.floor(x).astype(jnp.int32)
    y_hi = y_low >= H - 1
    x_hi = x_low >= W - 1
    y_low = jnp.where(y_hi, H - 1, y_low)
    x_low = jnp.where(x_hi, W - 1, x_low)
    y_high = jnp.where(y_hi, H - 1, y_low + 1)
    x_high = jnp.where(x_hi, W - 1, x_low + 1)
    y = jnp.where(y_hi, y_low.astype(jnp.float32), y)
    x = jnp.where(x_hi, x_low.astype(jnp.float32), x)
    ly = y - y_low.astype(jnp.float32)
    lx = x - x_low.astype(jnp.float32)
    hy = 1.0 - ly
    hx = 1.0 - lx

    wy = jnp.where(corner < 2, hy, ly)
    wx = jnp.where(corner % 2 == 0, hx, lx)
    ysel = jnp.where(corner < 2, y_low, y_high)
    xsel = jnp.where(corner % 2 == 0, x_low, x_high)

    idx = bidx * (H * W) + ysel * W + xsel
    wgt = jnp.where(valid, wy * wx * (1.0 / (_SN * _SN)), 0.0)
    return idx, wgt


def _prep(rois, H, W):
    N = rois.shape[0]
    nblk = 64

    def body(rois_ref, idx_ref, wgt_ref):
        idx, wgt = _prep_math(rois_ref[...], H, W)
        idx_ref[...] = idx
        wgt_ref[...] = wgt

    return pl.pallas_call(
        body,
        grid=(N // nblk,),
        in_specs=[pl.BlockSpec((nblk, 6), lambda i: (i, 0))],
        out_specs=[
            pl.BlockSpec((nblk, _PB * _E), lambda i: (i, 0)),
            pl.BlockSpec((nblk, _PB * _E), lambda i: (i, 0)),
        ],
        out_shape=[
            jax.ShapeDtypeStruct((N, _PB * _E), jnp.int32),
            jax.ShapeDtypeStruct((N, _PB * _E), jnp.float32),
        ],
    )(rois)


def _sc_pool(table, idx_flat, wgt_flat, tot_bins):
    C = table.shape[1] * 2
    bins_w = tot_bins // _NW
    n_chunks = bins_w // _CH_BINS
    nvec = C // 16

    ew = bins_w * _E

    mesh = plsc.VectorSubcoreMesh(core_axis_name="c", subcore_axis_name="s")

    @functools.partial(
        pl.kernel,
        mesh=mesh,
        out_type=jax.ShapeDtypeStruct((tot_bins, C), jnp.float32),
        scratch_types=[
            pltpu.VMEM((ew,), jnp.int32),
            pltpu.VMEM((ew,), jnp.float32),
            pltpu.VMEM((_ROWS, C // 2), jnp.int32),
            pltpu.VMEM((_ROWS, C // 2), jnp.int32),
            pltpu.VMEM((_ROWS, C // 2), jnp.int32),
            pltpu.VMEM((_CH_BINS, C), jnp.float32),
            pltpu.SemaphoreType.DMA,
            pltpu.SemaphoreType.DMA,
            pltpu.SemaphoreType.DMA,
        ],
        compiler_params=pltpu.CompilerParams(needs_layout_passes=False),
    )
    def body(table_hbm, idx_hbm, wgt_hbm, out_hbm,
             idx_all, wgt_all, rows0, rows1, rows2, out_v, sem0, sem1, sem2):
        wid = lax.axis_index("s") * 2 + lax.axis_index("c")
        rows = (rows0, rows1, rows2)
        sems = (sem0, sem1, sem2)

        pltpu.sync_copy(idx_hbm.at[pl.ds(wid * ew, ew)], idx_all)
        pltpu.sync_copy(wgt_hbm.at[pl.ds(wid * ew, ew)], wgt_all)

        def start_gather(ci, b):
            pltpu.async_copy(
                table_hbm.at[idx_all.at[pl.ds(ci * _ROWS, _ROWS)]],
                rows[b], sems[b])

        def wait_gather(ci, b):
            pltpu.make_async_copy(
                table_hbm.at[idx_all.at[pl.ds(ci * _ROWS, _ROWS)]],
                rows[b], sems[b]).wait()

        def compute(ci, b):
            rows_v = rows[b]
            gbin = wid * bins_w + ci * _CH_BINS

            @plsc.parallel_loop(0, _CH_BINS, unroll=2)
            def bin_body(bb):
                rbase = bb * _E
                wv = wgt_all[pl.ds(ci * _ROWS + rbase, _E)]
                accs = [jnp.zeros((32,), jnp.bfloat16) for _ in range(nvec // 2)]
                for j in range(_E):
                    wf = jnp.broadcast_to(wv[j], (16,))
                    wb = plsc.pack(wf, wf, format=plsc.PackFormat.INTERLEAVED)
                    rr = rbase + j
                    for t in range(nvec // 2):
                        vb = plsc.bitcast(
                            rows_v[rr, pl.ds(t * 16, 16)], jnp.bfloat16)
                        accs[t] = accs[t] + wb * vb
                for t in range(nvec // 2):
                    ai = plsc.bitcast(accs[t], jnp.int32)
                    lo = plsc.bitcast(ai << 16, jnp.float32)
                    hi = plsc.bitcast(ai & jnp.int32(-65536), jnp.float32)
                    out_v[bb, pl.ds((2 * t) * 16, 16)] = lo
                    out_v[bb, pl.ds((2 * t + 1) * 16, 16)] = hi

            pltpu.sync_copy(out_v, out_hbm.at[pl.ds(gbin, _CH_BINS)])

        start_gather(0, 0)
        start_gather(1, 1)

        def outer(io, carry):
            for b in range(3):
                ci = io * 3 + b
                wait_gather(ci, b)
                start_gather(ci + 2, (b + 2) % 3)
                compute(ci, b)
            return carry

        n_main = (n_chunks - 4) // 3
        lax.fori_loop(0, n_main, outer, 0)
        for ci_ in range(n_main * 3, n_chunks):
            b = ci_ % 3
            wait_gather(ci_, b)
            if ci_ + 2 < n_chunks:
                start_gather(ci_ + 2, (ci_ + 2) % 3)
            compute(ci_, b)

    return body(table, idx_flat, wgt_flat)


def kernel(features, rois):
    B, C, H, W = features.shape
    N = rois.shape[0]
    tb = jnp.transpose(features, (0, 2, 3, 1)).reshape(
        B * H * W, C).astype(jnp.bfloat16)
    u = jax.lax.bitcast_convert_type(tb, jnp.uint16).reshape(-1, C // 32, 2, 16)
    ti = (u[:, :, 0, :].astype(jnp.int32)
          | (u[:, :, 1, :].astype(jnp.int32) << 16)).reshape(-1, C // 2)
    idx, wgt = _prep(rois, H, W)
    out = _sc_pool(ti, idx.reshape(-1), wgt.reshape(-1), N * _PB)
    return out.reshape(N, _PB, C).transpose(0, 2, 1).reshape(N, C, _OUT_H, _OUT_W)
